# Initial kernel scaffold; baseline (speedup 1.0000x reference)
import jax
import jax.numpy as jnp
from jax.experimental import pallas as pl


def kernel(user_dynamic, item_dynamic, item_static, skill, response_h_by, response_h_pby, item_nbr_users, user_nbr_items, W_user, W_item, knowledge_init, q1_W, q1_b, Lo_W, Lo_b, Lq_W, Lq_b, fo_W, fo_b, l1_W, l1_b, l2_W, l2_b, l3_W, l3_b, l4_W, l4_b):
    raise NotImplementedError("write your pallas kernel here")


# speedup vs baseline: 4.6942x; 4.6942x over previous
import functools

import jax
import jax.numpy as jnp
from jax import lax
from jax.experimental import pallas as pl
from jax.experimental.pallas import tpu as pltpu
from jax.experimental.pallas import tpu_sc as plsc

H = 128
U = 10000
I = 5000
LU = 20
LI = 32
E1 = I * LI
E2 = U * LU
RSQRT_H = 1.0 / float(H) ** 0.5

IB = 200
UB = 400
IBP = 1000
UBP = 2000


def _dot(a, b):
    return lax.dot_general(a, b, (((1,), (0,)), ((), ())),
                           preferred_element_type=jnp.float32)


def _item_pre_body(ist_ref, sk_ref, idyn_ref, l1W_ref, l1b_ref, l2W_ref,
                   l2b_ref, l3W_ref, l3b_ref, l4W_ref, l4b_ref, q1W_ref,
                   q1b_ref, Wit_ref, pk_ref, q_ref, pv_ref, idnq_ref):
    ia = (_dot(ist_ref[...], l1W_ref[:H]) + _dot(sk_ref[...], l1W_ref[H:])
          + l1b_ref[...])
    pk_ref[...] = _dot(ia, l2W_ref[H:2 * H]) + l2b_ref[...]
    q_ref[...] = _dot(ia, l3W_ref[...]) + l3b_ref[...]
    pv_ref[...] = _dot(ia, l4W_ref[:H]) + l4b_ref[...]
    idn = _dot(idyn_ref[...], Wit_ref[...])
    idnq_ref[...] = (_dot(idn, q1W_ref[:H]) + _dot(ia, q1W_ref[H:2 * H])
                     + q1b_ref[...])


def _ud2_body(ud_ref, Wu_ref, l2W_ref, out_ref):
    wc = _dot(Wu_ref[...], l2W_ref[:H])
    out_ref[...] = _dot(ud_ref[...], wc)


def _sc_gather_body(t1_hbm, t2_hbm, idx1_hbm, idx2_hbm, out1_hbm, out2_hbm,
                    idx_v, rows_v, idxt_v, rowst_v, sem):
    info = plsc.get_sparse_core_info()
    nw = info.num_cores * info.num_subcores
    wid = lax.axis_index("s") * info.num_cores + lax.axis_index("c")
    C = 128
    n1 = E1 // C
    n2 = E2 // C
    tail = E2 - n2 * C

    def body1(k, carry):
        chunk = wid + nw * k

        @pl.when(chunk < n1)
        def _():
            base = pl.multiple_of(chunk * C, C)
            pltpu.sync_copy(idx1_hbm.at[pl.ds(base, C)], idx_v)
            pltpu.async_copy(t1_hbm.at[idx_v], rows_v, sem).wait()
            pltpu.sync_copy(rows_v, out1_hbm.at[pl.ds(base, C)])
        return carry

    lax.fori_loop(0, (n1 + nw - 1) // nw, body1, 0)

    def body2(k, carry):
        chunk = wid + nw * k

        @pl.when(chunk < n2)
        def _():
            base = pl.multiple_of(chunk * C, C)
            pltpu.sync_copy(idx2_hbm.at[pl.ds(base, C)], idx_v)
            pltpu.async_copy(t2_hbm.at[idx_v], rows_v, sem).wait()
            pltpu.sync_copy(rows_v, out2_hbm.at[pl.ds(base, C)])

        @pl.when(chunk == n2)
        def _():
            pltpu.sync_copy(idx2_hbm.at[pl.ds(n2 * C, tail)], idxt_v)
            pltpu.async_copy(t2_hbm.at[idxt_v], rowst_v, sem).wait()
            pltpu.sync_copy(rowst_v, out2_hbm.at[pl.ds(n2 * C, tail)])
        return carry

    lax.fori_loop(0, (n2 + 1 + nw - 1) // nw, body2, 0)


def _attn_body(rh_ref, gk_ref, q_ref, pk_ref, pv_ref, l2W_ref, l4W_ref,
               out_ref):
    rh2 = rh_ref[...].reshape(IB * LI, H)
    k2 = gk_ref[...] + _dot(rh2, l2W_ref[2 * H:])
    q = q_ref[...]
    k3 = k2.reshape(IB, LI, H) + pk_ref[...][:, None, :]
    e = jnp.sum(k3 * q[:, None, :], axis=2) * RSQRT_H
    m = jnp.max(e, axis=1, keepdims=True)
    a = jnp.exp(e - m)
    al = a / jnp.sum(a, axis=1, keepdims=True)
    v3 = _dot(rh2, l4W_ref[H:]).reshape(IB, LI, H)
    out_ref[...] = pv_ref[...] + jnp.sum(al[:, :, None] * v3, axis=1)


def _rec_body(g_ref, rp_ref, q1W_ref, LqW_ref, Lqb_ref, LoW_ref, Lob_ref,
              foW_ref, fob_ref, kin_ref, out_ref):
    q1C = q1W_ref[2 * H:]
    LqW = LqW_ref[...]
    Lqb = Lqb_ref[...]
    LoA = LoW_ref[:H]
    LoB = LoW_ref[H:]
    Lob = Lob_ref[...]
    foA = foW_ref[:H]
    foB = foW_ref[H:]
    fob = fob_ref[...]
    kkk = jnp.broadcast_to(kin_ref[...], (UB, H))
    for t in range(LU):
        gt = g_ref[t]
        rt = rp_ref[:, t, :]
        q1 = gt + _dot(kkk, q1C)
        xt = jnp.tanh(_dot(q1, LqW) + Lqb)
        xx = jax.nn.sigmoid(_dot(q1, LoA) + _dot(rt, LoB) + Lob) * xt
        foin = jax.nn.sigmoid(_dot(rt, foA) + _dot(kkk, foB) + fob)
        kkk = foin * kkk + (1.0 - foin) * xx
    out_ref[...] = kkk


def _row2(bs):
    return pl.BlockSpec(bs, lambda i: (i, 0))


def _whole2(shape):
    return pl.BlockSpec(shape, lambda i: (0, 0))


def kernel(user_dynamic, item_dynamic, item_static, skill, response_h_by,
           response_h_pby, item_nbr_users, user_nbr_items, W_user, W_item,
           knowledge_init, q1_W, q1_b, Lo_W, Lo_b, Lq_W, Lq_b, fo_W, fo_b,
           l1_W, l1_b, l2_W, l2_b, l3_W, l3_b, l4_W, l4_b):
    f32 = jnp.float32
    l1b = l1_b.reshape(1, H)
    l2b = l2_b.reshape(1, H)
    l3b = l3_b.reshape(1, H)
    l4b = l4_b.reshape(1, H)
    q1b = q1_b.reshape(1, H)
    Lqb = Lq_b.reshape(1, H)
    Lob = Lo_b.reshape(1, H)
    fob = fo_b.reshape(1, H)

    pk, q, pv, idnq = pl.pallas_call(
        _item_pre_body,
        grid=(I // IBP,),
        in_specs=[
            _row2((IBP, H)), _row2((IBP, H)), _row2((IBP, H)),
            _whole2((2 * H, H)), _whole2((1, H)),
            _whole2((3 * H, H)), _whole2((1, H)),
            _whole2((H, H)), _whole2((1, H)),
            _whole2((2 * H, H)), _whole2((1, H)),
            _whole2((3 * H, H)), _whole2((1, H)),
            _whole2((H, H)),
        ],
        out_specs=[_row2((IBP, H))] * 4,
        out_shape=[jax.ShapeDtypeStruct((I, H), f32)] * 4,
    )(item_static, skill, item_dynamic, l1_W, l1b, l2_W, l2b, l3_W, l3b,
      l4_W, l4b, q1_W, q1b, W_item)

    ud2 = pl.pallas_call(
        _ud2_body,
        grid=(U // UBP,),
        in_specs=[_row2((UBP, H)), _whole2((H, H)), _whole2((3 * H, H))],
        out_specs=_row2((UBP, H)),
        out_shape=jax.ShapeDtypeStruct((U, H), f32),
    )(user_dynamic, W_user, l2_W)

    idx1 = item_nbr_users.reshape(E1)
    idx2 = jnp.swapaxes(user_nbr_items, 0, 1).reshape(E2)
    mesh = plsc.VectorSubcoreMesh(core_axis_name="c", subcore_axis_name="s")
    gk, g2 = pl.kernel(
        _sc_gather_body,
        out_type=[jax.ShapeDtypeStruct((E1, H), f32),
                  jax.ShapeDtypeStruct((E2, H), f32)],
        mesh=mesh,
        scratch_types=[pltpu.VMEM((128,), jnp.int32),
                       pltpu.VMEM((128, H), f32),
                       pltpu.VMEM((64,), jnp.int32),
                       pltpu.VMEM((64, H), f32),
                       pltpu.SemaphoreType.DMA],
    )(ud2, idnq, idx1, idx2)

    item_out = pl.pallas_call(
        _attn_body,
        grid=(I // IB,),
        in_specs=[
            pl.BlockSpec((IB, LI, H), lambda i: (i, 0, 0)),
            _row2((IB * LI, H)),
            _row2((IB, H)), _row2((IB, H)), _row2((IB, H)),
            _whole2((3 * H, H)), _whole2((2 * H, H)),
        ],
        out_specs=_row2((IB, H)),
        out_shape=jax.ShapeDtypeStruct((I, H), f32),
    )(response_h_by, gk, q, pk, pv, l2_W, l4_W)

    g3 = g2.reshape(LU, U, H)
    user_out = pl.pallas_call(
        _rec_body,
        grid=(U // UB,),
        in_specs=[
            pl.BlockSpec((LU, UB, H), lambda i: (0, i, 0)),
            pl.BlockSpec((UB, LU, H), lambda i: (i, 0, 0)),
            _whole2((3 * H, H)),
            _whole2((H, H)), _whole2((1, H)),
            _whole2((2 * H, H)), _whole2((1, H)),
            _whole2((2 * H, H)), _whole2((1, H)),
            _whole2((1, H)),
        ],
        out_specs=_row2((UB, H)),
        out_shape=jax.ShapeDtypeStruct((U, H), f32),
    )(g3, response_h_pby, q1_W, Lq_W, Lqb, Lo_W, Lob, fo_W, fob,
      knowledge_init)

    return user_out, item_out
